# baseline (speedup 1.0000x reference)
import functools

import jax
import jax.numpy as jnp
from jax import lax
from jax.experimental import pallas as pl
from jax.experimental.pallas import tpu as pltpu
from jax.experimental.pallas import tpu_sc as plsc

BATCH = 16384
FIELDS = 26
EMBEDDING_SIZE = 32
CARD = 1000000

NC = 2
NS = 16
NW = NC * NS

D = EMBEDDING_SIZE
BPW = BATCH // NW
L = 16
CHB = 16
NCHUNK = BPW // CHB
NBUF = 4
assert BPW * NW == BATCH and (NCHUNK - 4) % 2 == 0


def _detile_kernel(idxt_hbm, out_hbm, idxt_v, obuf_v, lanes_f):
    wid = lax.axis_index("s") * NC + lax.axis_index("c")
    b0 = wid * BPW
    pltpu.sync_copy(idxt_hbm.at[:, pl.ds(b0, BPW)], idxt_v)
    lanes = lax.iota(jnp.int32, L)

    def fcol(f, _):
        fv = jnp.full((L,), 0, jnp.int32) + f

        def grp(g, _):
            o = g * L
            v = idxt_v[f, pl.ds(o, L)]
            plsc.store_scatter(obuf_v, [lanes + o, fv], v)
            return 0

        lax.fori_loop(0, BPW // L, grp, 0)
        return 0

    lax.fori_loop(0, FIELDS, fcol, 0)
    pltpu.sync_copy(obuf_v, out_hbm.at[pl.ds(b0, BPW)])


def _gather_kernel(idxb_hbm, w_hbm, out_hbm, idxb_v, rows_v,
                   gsem0, gsem1, osem0, osem1):
    wid = lax.axis_index("s") * NC + lax.axis_index("c")
    b0 = wid * BPW
    pltpu.sync_copy(idxb_hbm.at[pl.ds(b0, BPW)], idxb_v)

    gsems = (gsem0, gsem1)
    osems = (osem0, osem1)

    def _gathers(j, slot, par):
        return [pltpu.make_async_copy(
            w_hbm.at[idxb_v.at[j * CHB + i]],
            rows_v.at[slot, i], gsems[par]) for i in range(CHB)]

    class gather_chunk:
        def __init__(self, j, slot, par):
            self.cs = _gathers(j, slot, par)

        def start(self):
            for c in self.cs:
                c.start()

        def wait(self):
            for c in self.cs:
                c.wait()

    def out_chunk(j, slot, par):
        return pltpu.make_async_copy(
            rows_v.at[slot],
            out_hbm.at[pl.ds(b0 + j * CHB, CHB)], osems[par])

    gather_chunk(0, 0, 0).start()
    gather_chunk(1, 1, 1).start()

    for j in (0, 1):
        gather_chunk(j, j, j % 2).wait()
        gather_chunk(j + 2, j + 2, j % 2).start()
        out_chunk(j, j, j % 2).start()

    def step(j, par):
        slot = lax.rem(j, NBUF)
        gather_chunk(j, slot, par).wait()
        out_chunk(j - 2, lax.rem(j - 2, NBUF), par).wait()
        gather_chunk(j + 2, lax.rem(j + 2, NBUF), par).start()
        out_chunk(j, slot, par).start()

    def body(i, _):
        j = 2 + 2 * i
        step(j, 0)
        step(j + 1, 1)
        return 0

    lax.fori_loop(0, (NCHUNK - 4) // 2, body, 0)

    for j in (NCHUNK - 2, NCHUNK - 1):
        gather_chunk(j, j % NBUF, j % 2).wait()
        out_chunk(j - 2, (j - 2) % NBUF, j % 2).wait()
        out_chunk(j, j % NBUF, j % 2).start()
    for j in (NCHUNK - 2, NCHUNK - 1):
        out_chunk(j, j % NBUF, j % 2).wait()


@jax.jit
def kernel(inputs, weight):
    idxt = inputs.astype(jnp.int32).T
    mesh = plsc.VectorSubcoreMesh(core_axis_name="c", subcore_axis_name="s")
    idxb = pl.kernel(
        _detile_kernel,
        out_type=jax.ShapeDtypeStruct((BATCH, FIELDS), jnp.int32),
        mesh=mesh,
        scratch_types=[
            pltpu.VMEM((FIELDS, BPW), jnp.int32),
            pltpu.VMEM((BPW, FIELDS), jnp.int32),
            pltpu.SMEM((1,), jnp.int32),
        ],
        compiler_params=pltpu.CompilerParams(needs_layout_passes=False),
    )(idxt)
    out = pl.kernel(
        _gather_kernel,
        out_type=jax.ShapeDtypeStruct((BATCH, FIELDS, D), jnp.float32),
        mesh=mesh,
        scratch_types=[
            pltpu.VMEM((BPW, FIELDS), jnp.int32),
            pltpu.VMEM((NBUF, CHB, FIELDS, D), jnp.float32),
            pltpu.SemaphoreType.DMA,
            pltpu.SemaphoreType.DMA,
            pltpu.SemaphoreType.DMA,
            pltpu.SemaphoreType.DMA,
        ],
        compiler_params=pltpu.CompilerParams(use_tc_tiling_on_sc=False),
    )(idxb, weight)
    return out
